# baseline (speedup 1.0000x reference)
import functools

import jax
import jax.numpy as jnp
from jax import lax
from jax.experimental import pallas as pl
from jax.experimental.pallas import tpu as pltpu
from jax.experimental.pallas import tpu_sc as plsc

NC = 2
NS = 16
NW = NC * NS
LANE = 16
CHUNK = 128


def _mm_gh_body(h_ref, wm_ref, whh_ref, bhh_ref, m_ref, gh_ref):
    h = h_ref[...]
    m_ref[...] = jnp.dot(h, wm_ref[...], preferred_element_type=jnp.float32)
    gh_ref[...] = (
        jnp.dot(h, whh_ref[...], preferred_element_type=jnp.float32)
        + bhh_ref[...]
    )


def _mm_gh(h, wm, whhT, bhh2, row_block):
    n, c = h.shape
    grid = n // row_block
    return pl.pallas_call(
        _mm_gh_body,
        grid=(grid,),
        in_specs=[
            pl.BlockSpec((row_block, c), lambda i: (i, 0)),
            pl.BlockSpec((c, c), lambda i: (0, 0)),
            pl.BlockSpec((c, 3 * c), lambda i: (0, 0)),
            pl.BlockSpec((1, 3 * c), lambda i: (0, 0)),
        ],
        out_specs=[
            pl.BlockSpec((row_block, c), lambda i: (i, 0)),
            pl.BlockSpec((row_block, 3 * c), lambda i: (i, 0)),
        ],
        out_shape=[
            jax.ShapeDtypeStruct((n, c), jnp.float32),
            jax.ShapeDtypeStruct((n, 3 * c), jnp.float32),
        ],
    )(h, wm, whhT, bhh2)


def _gru_body(p_ref, gh_ref, h_ref, wih_ref, bih_ref, o_ref):
    c = h_ref.shape[1]
    agg = p_ref[0]
    for k in range(1, p_ref.shape[0]):
        agg = agg + p_ref[k]
    gi = (
        jnp.dot(agg, wih_ref[...], preferred_element_type=jnp.float32)
        + bih_ref[...]
    )
    gh = gh_ref[...]
    h = h_ref[...]
    r = jax.nn.sigmoid(gi[:, :c] + gh[:, :c])
    z = jax.nn.sigmoid(gi[:, c:2 * c] + gh[:, c:2 * c])
    n = jnp.tanh(gi[:, 2 * c:] + r * gh[:, 2 * c:])
    o_ref[...] = (1.0 - z) * n + z * h


def _gru(parts, gh, h, wihT, bih2, row_block):
    n, c = h.shape
    grid = n // row_block
    return pl.pallas_call(
        _gru_body,
        grid=(grid,),
        in_specs=[
            pl.BlockSpec((parts.shape[0], row_block, c), lambda i: (0, i, 0)),
            pl.BlockSpec((row_block, 3 * c), lambda i: (i, 0)),
            pl.BlockSpec((row_block, c), lambda i: (i, 0)),
            pl.BlockSpec((c, 3 * c), lambda i: (0, 0)),
            pl.BlockSpec((1, 3 * c), lambda i: (0, 0)),
        ],
        out_specs=pl.BlockSpec((row_block, c), lambda i: (i, 0)),
        out_shape=jax.ShapeDtypeStruct((n, c), jnp.float32),
    )(parts, gh, h, wihT, bih2)


@functools.cache
def _make_sc_agg(n_nodes, c, cpt0, cpt1):
    agg_rows = -(-(n_nodes + 1) // (NS * 8)) * (NS * 8)
    zps = agg_rows // NS
    assert cpt0 % 8 == 0 and cpt1 % 8 == 0
    piece = 48
    ncor = NC
    mesh = plsc.VectorSubcoreMesh(
        core_axis_name="c", subcore_axis_name="s",
        num_cores=ncor, num_subcores=NS)

    @functools.partial(
        pl.kernel,
        mesh=mesh,
        out_type=jax.ShapeDtypeStruct((ncor, agg_rows, c), jnp.float32),
        scratch_types=[
            pltpu.VMEM((piece, CHUNK), jnp.int32),
            pltpu.VMEM((piece, CHUNK), jnp.int32),
            pltpu.VMEM((2, CHUNK, c), jnp.float32),
            pltpu.VMEM_SHARED((agg_rows, c), jnp.float32),
            pltpu.SemaphoreType.DMA,
            pltpu.SemaphoreType.DMA,
        ],
    )
    def sc_agg(m_hbm, src_hbm, dst_hbm, out_hbm,
               src_v, dst_v, rows_v, agg_sh, sem0, sem1):
        core = lax.axis_index("c")
        sub = lax.axis_index("s")

        @pl.loop(0, CHUNK)
        def _(r):
            for j in range(c // LANE):
                rows_v[0, r, pl.ds(j * LANE, LANE)] = jnp.zeros(
                    (LANE,), jnp.float32)

        zfull, zrem = zps // CHUNK, zps % CHUNK

        @pl.loop(0, zfull)
        def _(k):
            pltpu.sync_copy(rows_v.at[0],
                            agg_sh.at[pl.ds(sub * zps + k * CHUNK, CHUNK)])
        if zrem:
            pltpu.sync_copy(
                rows_v.at[0, pl.ds(0, zrem)],
                agg_sh.at[pl.ds(sub * zps + zfull * CHUNK, zrem)])
        plsc.subcore_barrier()

        def tile_work(cpt_c, tile_base):
            done = 0
            while done < cpt_c:
                pc = min(piece, cpt_c - done)
                cbase = tile_base + done
                done += pc
                pltpu.sync_copy(src_hbm.at[pl.ds(cbase, pc)],
                                src_v.at[pl.ds(0, pc)])
                pltpu.sync_copy(dst_hbm.at[pl.ds(cbase, pc)],
                                dst_v.at[pl.ds(0, pc)])

                pltpu.async_copy(m_hbm.at[src_v.at[0]], rows_v.at[0], sem0)

                @pl.loop(0, pc, step=2)
                def _(j2):
                    pltpu.async_copy(
                        m_hbm.at[src_v.at[j2 + 1]], rows_v.at[1], sem1)
                    pltpu.make_async_copy(
                        m_hbm.at[src_v.at[j2]], rows_v.at[0], sem0).wait()
                    pltpu.sync_copy(
                        rows_v.at[0], agg_sh.at[dst_v.at[j2]], add=True)

                    @pl.when(j2 + 2 < pc)
                    def _():
                        pltpu.async_copy(
                            m_hbm.at[src_v.at[j2 + 2]], rows_v.at[0], sem0)

                    pltpu.make_async_copy(
                        m_hbm.at[src_v.at[j2 + 1]], rows_v.at[1], sem1).wait()
                    pltpu.sync_copy(
                        rows_v.at[1], agg_sh.at[dst_v.at[j2 + 1]], add=True)

        if cpt0:
            @pl.when(core == 0)
            def _():
                tile_work(cpt0, sub * cpt0)

        if cpt1:
            @pl.when(core == 1)
            def _():
                tile_work(cpt1, NS * cpt0 + sub * cpt1)

        plsc.subcore_barrier()
        pltpu.sync_copy(agg_sh.at[pl.ds(sub * zps, zps)],
                        out_hbm.at[core, pl.ds(sub * zps, zps)])

    return sc_agg


def kernel(x, edge_index, weight, W_ih, W_hh, b_ih, b_hh):
    n, c = x.shape
    e = edge_index.shape[1]
    num_layers = weight.shape[0]

    src = edge_index[0].astype(jnp.int32)
    dst = edge_index[1].astype(jnp.int32)
    pair_cpt = -(-e // (NS * CHUNK * 16)) * 16
    cpt0 = min(max(int(round(pair_cpt * 0.85 / 8)) * 8, 8), pair_cpt - 8)
    cpt1 = pair_cpt - cpt0
    e_pad = NS * CHUNK * pair_cpt
    src_p = jnp.concatenate(
        [src, jnp.zeros((e_pad - e,), jnp.int32)]).reshape(NS * pair_cpt, CHUNK)
    dst_p = jnp.concatenate(
        [dst, jnp.full((e_pad - e,), n, jnp.int32)]).reshape(NS * pair_cpt, CHUNK)

    whhT = W_hh.T
    wihT = W_ih.T
    bhh2 = b_hh.reshape(1, -1)
    bih2 = b_ih.reshape(1, -1)

    sc_agg = _make_sc_agg(n, c, cpt0, cpt1)
    row_block = 1000 if n % 1000 == 0 else n

    h = x
    for i in range(num_layers):
        m, gh = _mm_gh(h, weight[i], whhT, bhh2, row_block)
        parts = sc_agg(m, src_p, dst_p)
        h = _gru(parts, gh, h, wihT, bih2, row_block)
    return h


# speedup vs baseline: 1.1137x; 1.1137x over previous
import functools

import jax
import jax.numpy as jnp
from jax import lax
from jax.experimental import pallas as pl
from jax.experimental.pallas import tpu as pltpu
from jax.experimental.pallas import tpu_sc as plsc

NC = 2
NS = 16
NW = NC * NS
LANE = 16
CHUNK = 128


def _mm_gh_body(h_ref, wm_ref, whh_ref, bhh_ref, m_ref, gh_ref):
    h = h_ref[...]
    m_ref[...] = jnp.dot(h, wm_ref[...], preferred_element_type=jnp.float32)
    gh_ref[...] = (
        jnp.dot(h, whh_ref[...], preferred_element_type=jnp.float32)
        + bhh_ref[...]
    )


def _mm_gh(h, wm, whhT, bhh2, row_block):
    n, c = h.shape
    grid = n // row_block
    return pl.pallas_call(
        _mm_gh_body,
        grid=(grid,),
        in_specs=[
            pl.BlockSpec((row_block, c), lambda i: (i, 0)),
            pl.BlockSpec((c, c), lambda i: (0, 0)),
            pl.BlockSpec((c, 3 * c), lambda i: (0, 0)),
            pl.BlockSpec((1, 3 * c), lambda i: (0, 0)),
        ],
        out_specs=[
            pl.BlockSpec((row_block, c), lambda i: (i, 0)),
            pl.BlockSpec((row_block, 3 * c), lambda i: (i, 0)),
        ],
        out_shape=[
            jax.ShapeDtypeStruct((n, c), jnp.float32),
            jax.ShapeDtypeStruct((n, 3 * c), jnp.float32),
        ],
    )(h, wm, whhT, bhh2)


def _gru_body(p_ref, gh_ref, h_ref, wih_ref, bih_ref, o_ref):
    c = h_ref.shape[1]
    agg = p_ref[0]
    for k in range(1, p_ref.shape[0]):
        agg = agg + p_ref[k]
    gi = (
        jnp.dot(agg, wih_ref[...], preferred_element_type=jnp.float32)
        + bih_ref[...]
    )
    gh = gh_ref[...]
    h = h_ref[...]
    r = jax.nn.sigmoid(gi[:, :c] + gh[:, :c])
    z = jax.nn.sigmoid(gi[:, c:2 * c] + gh[:, c:2 * c])
    n = jnp.tanh(gi[:, 2 * c:] + r * gh[:, 2 * c:])
    o_ref[...] = (1.0 - z) * n + z * h


def _gru(parts, gh, h, wihT, bih2, row_block):
    n, c = h.shape
    grid = n // row_block
    return pl.pallas_call(
        _gru_body,
        grid=(grid,),
        in_specs=[
            pl.BlockSpec((parts.shape[0], row_block, c), lambda i: (0, i, 0)),
            pl.BlockSpec((row_block, 3 * c), lambda i: (i, 0)),
            pl.BlockSpec((row_block, c), lambda i: (i, 0)),
            pl.BlockSpec((c, 3 * c), lambda i: (0, 0)),
            pl.BlockSpec((1, 3 * c), lambda i: (0, 0)),
        ],
        out_specs=pl.BlockSpec((row_block, c), lambda i: (i, 0)),
        out_shape=jax.ShapeDtypeStruct((n, c), jnp.float32),
    )(parts, gh, h, wihT, bih2)


@functools.cache
def _make_sc_agg(n_nodes, c, cpt0, cpt1):
    agg_rows = -(-(n_nodes + 1) // (NS * 8)) * (NS * 8)
    zps = agg_rows // NS
    assert cpt0 % 8 == 0 and cpt1 % 8 == 0
    piece = 48
    ncor = NC
    mesh = plsc.VectorSubcoreMesh(
        core_axis_name="c", subcore_axis_name="s",
        num_cores=ncor, num_subcores=NS)

    @functools.partial(
        pl.kernel,
        mesh=mesh,
        out_type=jax.ShapeDtypeStruct((ncor, agg_rows, c), jnp.float32),
        scratch_types=[
            pltpu.VMEM((piece, CHUNK), jnp.int32),
            pltpu.VMEM((piece, CHUNK), jnp.int32),
            pltpu.VMEM((2, CHUNK, c), jnp.float32),
            pltpu.VMEM_SHARED((agg_rows, c), jnp.float32),
            pltpu.SemaphoreType.DMA,
            pltpu.SemaphoreType.DMA,
        ],
    )
    def sc_agg(m_hbm, src_hbm, dst_hbm, out_hbm,
               src_v, dst_v, rows_v, agg_sh, sem0, sem1):
        core = lax.axis_index("c")
        sub = lax.axis_index("s")

        @pl.loop(0, CHUNK)
        def _(r):
            for j in range(c // LANE):
                rows_v[0, r, pl.ds(j * LANE, LANE)] = jnp.zeros(
                    (LANE,), jnp.float32)

        zfull, zrem = zps // CHUNK, zps % CHUNK

        @pl.loop(0, zfull)
        def _(k):
            pltpu.sync_copy(rows_v.at[0],
                            agg_sh.at[pl.ds(sub * zps + k * CHUNK, CHUNK)])
        if zrem:
            pltpu.sync_copy(
                rows_v.at[0, pl.ds(0, zrem)],
                agg_sh.at[pl.ds(sub * zps + zfull * CHUNK, zrem)])
        plsc.subcore_barrier()

        def tile_work(cpt_c, tile_base):
            done = 0
            while done < cpt_c:
                pc = min(piece, cpt_c - done)
                cbase = tile_base + done
                done += pc
                pltpu.sync_copy(src_hbm.at[pl.ds(cbase, pc)],
                                src_v.at[pl.ds(0, pc)])
                pltpu.sync_copy(dst_hbm.at[pl.ds(cbase, pc)],
                                dst_v.at[pl.ds(0, pc)])

                pltpu.async_copy(m_hbm.at[src_v.at[0]], rows_v.at[0], sem0)

                @pl.loop(0, pc, step=2)
                def _(j2):
                    pltpu.async_copy(
                        m_hbm.at[src_v.at[j2 + 1]], rows_v.at[1], sem1)
                    pltpu.make_async_copy(
                        m_hbm.at[src_v.at[j2]], rows_v.at[0], sem0).wait()
                    pltpu.sync_copy(
                        rows_v.at[0], agg_sh.at[dst_v.at[j2]], add=True)

                    @pl.when(j2 + 2 < pc)
                    def _():
                        pltpu.async_copy(
                            m_hbm.at[src_v.at[j2 + 2]], rows_v.at[0], sem0)

                    pltpu.make_async_copy(
                        m_hbm.at[src_v.at[j2 + 1]], rows_v.at[1], sem1).wait()
                    pltpu.sync_copy(
                        rows_v.at[1], agg_sh.at[dst_v.at[j2 + 1]], add=True)

        if cpt0:
            @pl.when(core == 0)
            def _():
                tile_work(cpt0, sub * cpt0)

        if cpt1:
            @pl.when(core == 1)
            def _():
                tile_work(cpt1, NS * cpt0 + sub * cpt1)

        plsc.subcore_barrier()
        pltpu.sync_copy(agg_sh.at[pl.ds(sub * zps, zps)],
                        out_hbm.at[core, pl.ds(sub * zps, zps)])

    return sc_agg


def kernel(x, edge_index, weight, W_ih, W_hh, b_ih, b_hh):
    n, c = x.shape
    e = edge_index.shape[1]
    num_layers = weight.shape[0]

    src = edge_index[0].astype(jnp.int32)
    dst = edge_index[1].astype(jnp.int32)
    pair_cpt = -(-e // (NS * CHUNK * 16)) * 16
    cpt0 = min(max(int(round(pair_cpt * 0.9 / 16)) * 16, 16), pair_cpt - 16)
    cpt1 = pair_cpt - cpt0
    e_pad = NS * CHUNK * pair_cpt
    src_p = jnp.concatenate(
        [src, jnp.zeros((e_pad - e,), jnp.int32)]).reshape(NS * pair_cpt, CHUNK)
    dst_p = jnp.concatenate(
        [dst, jnp.full((e_pad - e,), n, jnp.int32)]).reshape(NS * pair_cpt, CHUNK)

    whhT = W_hh.T
    wihT = W_ih.T
    bhh2 = b_hh.reshape(1, -1)
    bih2 = b_ih.reshape(1, -1)

    sc_agg = _make_sc_agg(n, c, cpt0, cpt1)
    row_block = 1000 if n % 1000 == 0 else n

    h = x
    for i in range(num_layers):
        m, gh = _mm_gh(h, weight[i], whhT, bhh2, row_block)
        parts = sc_agg(m, src_p, dst_p)
        h = _gru(parts, gh, h, wihT, bih2, row_block)
    return h


# speedup vs baseline: 1.1205x; 1.0060x over previous
import functools

import jax
import jax.numpy as jnp
from jax import lax
from jax.experimental import pallas as pl
from jax.experimental.pallas import tpu as pltpu
from jax.experimental.pallas import tpu_sc as plsc

NC = 2
NS = 16
NW = NC * NS
LANE = 16
CHUNK = 128


def _mm_body(h_ref, wm_ref, m_ref):
    m_ref[...] = jnp.dot(h_ref[...], wm_ref[...],
                         preferred_element_type=jnp.float32)


def _mm(h, wm, row_block):
    n, c = h.shape
    return pl.pallas_call(
        _mm_body,
        grid=(n // row_block,),
        in_specs=[
            pl.BlockSpec((row_block, c), lambda i: (i, 0)),
            pl.BlockSpec((c, c), lambda i: (0, 0)),
        ],
        out_specs=pl.BlockSpec((row_block, c), lambda i: (i, 0)),
        out_shape=jax.ShapeDtypeStruct((n, c), jnp.float32),
    )(h, wm)


def _gh_body(h_ref, whh_ref, bhh_ref, gh_ref):
    gh_ref[...] = (
        jnp.dot(h_ref[...], whh_ref[...], preferred_element_type=jnp.float32)
        + bhh_ref[...]
    )


def _gh(h, whhT, bhh2, row_block):
    n, c = h.shape
    return pl.pallas_call(
        _gh_body,
        grid=(n // row_block,),
        in_specs=[
            pl.BlockSpec((row_block, c), lambda i: (i, 0)),
            pl.BlockSpec((c, 3 * c), lambda i: (0, 0)),
            pl.BlockSpec((1, 3 * c), lambda i: (0, 0)),
        ],
        out_specs=pl.BlockSpec((row_block, 3 * c), lambda i: (i, 0)),
        out_shape=jax.ShapeDtypeStruct((n, 3 * c), jnp.float32),
    )(h, whhT, bhh2)


def _gru_body(p_ref, gh_ref, h_ref, wih_ref, bih_ref, o_ref):
    c = h_ref.shape[1]
    agg = p_ref[0]
    for k in range(1, p_ref.shape[0]):
        agg = agg + p_ref[k]
    gi = (
        jnp.dot(agg, wih_ref[...], preferred_element_type=jnp.float32)
        + bih_ref[...]
    )
    gh = gh_ref[...]
    h = h_ref[...]
    r = jax.nn.sigmoid(gi[:, :c] + gh[:, :c])
    z = jax.nn.sigmoid(gi[:, c:2 * c] + gh[:, c:2 * c])
    n = jnp.tanh(gi[:, 2 * c:] + r * gh[:, 2 * c:])
    o_ref[...] = (1.0 - z) * n + z * h


def _gru(parts, gh, h, wihT, bih2, row_block):
    n, c = h.shape
    grid = n // row_block
    return pl.pallas_call(
        _gru_body,
        grid=(grid,),
        in_specs=[
            pl.BlockSpec((parts.shape[0], row_block, c), lambda i: (0, i, 0)),
            pl.BlockSpec((row_block, 3 * c), lambda i: (i, 0)),
            pl.BlockSpec((row_block, c), lambda i: (i, 0)),
            pl.BlockSpec((c, 3 * c), lambda i: (0, 0)),
            pl.BlockSpec((1, 3 * c), lambda i: (0, 0)),
        ],
        out_specs=pl.BlockSpec((row_block, c), lambda i: (i, 0)),
        out_shape=jax.ShapeDtypeStruct((n, c), jnp.float32),
    )(parts, gh, h, wihT, bih2)


@functools.cache
def _make_sc_agg(n_nodes, c, cpt0, cpt1):
    agg_rows = -(-(n_nodes + 1) // (NS * 8)) * (NS * 8)
    zps = agg_rows // NS
    assert cpt0 % 8 == 0 and cpt1 % 8 == 0
    piece = 48
    ncor = NC
    mesh = plsc.VectorSubcoreMesh(
        core_axis_name="c", subcore_axis_name="s",
        num_cores=ncor, num_subcores=NS)

    @functools.partial(
        pl.kernel,
        mesh=mesh,
        out_type=jax.ShapeDtypeStruct((ncor, agg_rows, c), jnp.float32),
        scratch_types=[
            pltpu.VMEM((piece, CHUNK), jnp.int32),
            pltpu.VMEM((piece, CHUNK), jnp.int32),
            pltpu.VMEM((2, CHUNK, c), jnp.float32),
            pltpu.VMEM_SHARED((agg_rows, c), jnp.float32),
            pltpu.SemaphoreType.DMA,
            pltpu.SemaphoreType.DMA,
        ],
    )
    def sc_agg(m_hbm, src_hbm, dst_hbm, out_hbm,
               src_v, dst_v, rows_v, agg_sh, sem0, sem1):
        core = lax.axis_index("c")
        sub = lax.axis_index("s")

        @pl.loop(0, CHUNK)
        def _(r):
            for j in range(c // LANE):
                rows_v[0, r, pl.ds(j * LANE, LANE)] = jnp.zeros(
                    (LANE,), jnp.float32)

        zfull, zrem = zps // CHUNK, zps % CHUNK

        @pl.loop(0, zfull)
        def _(k):
            pltpu.sync_copy(rows_v.at[0],
                            agg_sh.at[pl.ds(sub * zps + k * CHUNK, CHUNK)])
        if zrem:
            pltpu.sync_copy(
                rows_v.at[0, pl.ds(0, zrem)],
                agg_sh.at[pl.ds(sub * zps + zfull * CHUNK, zrem)])
        plsc.subcore_barrier()

        def tile_work(cpt_c, tile_base):
            done = 0
            while done < cpt_c:
                pc = min(piece, cpt_c - done)
                cbase = tile_base + done
                done += pc
                pltpu.sync_copy(src_hbm.at[pl.ds(cbase, pc)],
                                src_v.at[pl.ds(0, pc)])
                pltpu.sync_copy(dst_hbm.at[pl.ds(cbase, pc)],
                                dst_v.at[pl.ds(0, pc)])

                pltpu.async_copy(m_hbm.at[src_v.at[0]], rows_v.at[0], sem0)

                @pl.loop(0, pc, step=2)
                def _(j2):
                    pltpu.async_copy(
                        m_hbm.at[src_v.at[j2 + 1]], rows_v.at[1], sem1)
                    pltpu.make_async_copy(
                        m_hbm.at[src_v.at[j2]], rows_v.at[0], sem0).wait()
                    pltpu.sync_copy(
                        rows_v.at[0], agg_sh.at[dst_v.at[j2]], add=True)

                    @pl.when(j2 + 2 < pc)
                    def _():
                        pltpu.async_copy(
                            m_hbm.at[src_v.at[j2 + 2]], rows_v.at[0], sem0)

                    pltpu.make_async_copy(
                        m_hbm.at[src_v.at[j2 + 1]], rows_v.at[1], sem1).wait()
                    pltpu.sync_copy(
                        rows_v.at[1], agg_sh.at[dst_v.at[j2 + 1]], add=True)

        if cpt0:
            @pl.when(core == 0)
            def _():
                tile_work(cpt0, sub * cpt0)

        if cpt1:
            @pl.when(core == 1)
            def _():
                tile_work(cpt1, NS * cpt0 + sub * cpt1)

        plsc.subcore_barrier()
        pltpu.sync_copy(agg_sh.at[pl.ds(sub * zps, zps)],
                        out_hbm.at[core, pl.ds(sub * zps, zps)])

    return sc_agg


def kernel(x, edge_index, weight, W_ih, W_hh, b_ih, b_hh):
    n, c = x.shape
    e = edge_index.shape[1]
    num_layers = weight.shape[0]

    src = edge_index[0].astype(jnp.int32)
    dst = edge_index[1].astype(jnp.int32)
    pair_cpt = -(-e // (NS * CHUNK * 16)) * 16
    cpt0 = min(max(int(round(pair_cpt * 0.9 / 16)) * 16, 16), pair_cpt - 16)
    cpt1 = pair_cpt - cpt0
    e_pad = NS * CHUNK * pair_cpt
    src_p = jnp.concatenate(
        [src, jnp.zeros((e_pad - e,), jnp.int32)]).reshape(NS * pair_cpt, CHUNK)
    dst_p = jnp.concatenate(
        [dst, jnp.full((e_pad - e,), n, jnp.int32)]).reshape(NS * pair_cpt, CHUNK)

    whhT = W_hh.T
    wihT = W_ih.T
    bhh2 = b_hh.reshape(1, -1)
    bih2 = b_ih.reshape(1, -1)

    sc_agg = _make_sc_agg(n, c, cpt0, cpt1)
    row_block = 1000 if n % 1000 == 0 else n

    h = x
    for i in range(num_layers):
        m = _mm(h, weight[i], row_block)
        parts = sc_agg(m, src_p, dst_p)
        gh = _gh(h, whhT, bhh2, row_block)
        h = _gru(parts, gh, h, wihT, bih2, row_block)
    return h
